# baseline (speedup 1.0000x reference)
import functools

import jax
import jax.numpy as jnp
from jax import lax
from jax.experimental import pallas as pl
from jax.experimental.pallas import tpu as pltpu
from jax.experimental.pallas import tpu_sc as plsc

K = 8
B = 128


def _router_body(x_ref, rW_ref, rb_ref, er_ref, gate_ref, cnt_ref,
                 load_ref, z_ref, aux_ref, *, nblk, E, T, TB):
    i = pl.program_id(0)
    xb = x_ref[...]
    logits = jnp.dot(xb, rW_ref[...], preferred_element_type=jnp.float32) + rb_ref[...]
    mx = jnp.max(logits, axis=1, keepdims=True)
    ex = jnp.exp(logits - mx)
    se = jnp.sum(ex, axis=1, keepdims=True)
    probs = ex / se
    lse = mx + jnp.log(se)

    iota = lax.broadcasted_iota(jnp.int32, probs.shape, 1)
    work = probs
    sels, vals, ohs = [], [], []
    for _ in range(K):
        mj = jnp.max(work, axis=1, keepdims=True)
        ismax = work == mj
        sel = jnp.min(jnp.where(ismax, iota, E), axis=1, keepdims=True)
        onehot = iota == sel
        sels.append(sel)
        vals.append(mj)
        ohs.append(onehot)
        work = jnp.where(onehot, -jnp.inf, work)
    topv = jnp.concatenate(vals, axis=1)
    ssum = jnp.sum(topv, axis=1, keepdims=True)
    gates = topv / ssum
    gate_ref[...] = gates

    @pl.when(i == 0)
    def _init():
        cnt_ref[...] = jnp.zeros_like(cnt_ref)
        load_ref[...] = jnp.zeros_like(load_ref)
        z_ref[...] = jnp.zeros_like(z_ref)

    M = ohs[0].astype(jnp.float32)
    for oh in ohs[1:]:
        M = M + oh.astype(jnp.float32)
    ltri = (lax.broadcasted_iota(jnp.int32, (TB, TB), 0)
            > lax.broadcasted_iota(jnp.int32, (TB, TB), 1)).astype(jnp.float32)
    csum = jnp.dot(ltri, M, preferred_element_type=jnp.float32)
    tot = csum + cnt_ref[...]
    ranks = [jnp.sum(jnp.where(oh, tot, 0.0), axis=1, keepdims=True) for oh in ohs]
    rank_i = jnp.concatenate(ranks, axis=1).astype(jnp.int32)
    er_ref[...] = jnp.concatenate(sels, axis=1) * 65536 + rank_i

    cnt_ref[...] += jnp.sum(M, axis=0)[None, :]

    maskb = jnp.zeros_like(probs)
    for kk, oh in enumerate(ohs):
        maskb = maskb + jnp.where(oh, gates[:, kk:kk + 1], 0.0)
    load_ref[...] += jnp.sum(maskb, axis=0)[None, :]
    z_ref[...] += jnp.reshape(jnp.sum(lse * lse), (1, 1))

    @pl.when(i == nblk - 1)
    def _fin():
        load = load_ref[...] / T
        lb = 0.1 * jnp.sum((load - 1.0 / E) ** 2)
        aux_ref[...] = lb + 0.1 * z_ref[...] / T


def _expert_body(be_ref, xb_ref, vl_ref, xs_ref, gs_ref, W1_ref, b1_ref,
                 W2_ref, b2_ref, eo_ref, w1c_ref):
    i = pl.program_id(0)

    @pl.when((vl_ref[i] == 1)
             & ((i == 0) | (be_ref[i] != be_ref[jnp.maximum(i - 1, 0)])))
    def _cache():
        w1c_ref[...] = W1_ref[0].astype(jnp.bfloat16)

    @pl.when(vl_ref[i] == 0)
    def _zero():
        eo_ref[...] = jnp.zeros_like(eo_ref)

    @pl.when(vl_ref[i] == 1)
    def _run():
        xb = xs_ref[...].astype(jnp.bfloat16)
        h = jnp.maximum(
            jnp.dot(xb, w1c_ref[...], preferred_element_type=jnp.float32)
            + b1_ref[0], 0.0)
        w2 = W2_ref[0].astype(jnp.bfloat16)
        eo = jnp.dot(h.astype(jnp.bfloat16), w2,
                     preferred_element_type=jnp.float32) + b2_ref[0]
        eo = eo * gs_ref[...]
        pad = jnp.zeros((eo.shape[0], eo_ref.shape[1] - eo.shape[1]), jnp.float32)
        eo_ref[...] = jnp.concatenate([eo, pad], axis=1)


def kernel(x, rW, rb, W1, b1, W2, b2):
    T, D = x.shape
    E = rW.shape[1]
    H = W1.shape[2]
    C = W2.shape[2]
    C2 = 256
    TB = 256
    nblk = T // TB
    N = T * K
    MAXB = N // B + E
    NPAD = MAXB * B

    er, gate, cnt, _load, _z, aux = pl.pallas_call(
        functools.partial(_router_body, nblk=nblk, E=E, T=T, TB=TB),
        grid=(nblk,),
        in_specs=[
            pl.BlockSpec((TB, D), lambda i: (i, 0)),
            pl.BlockSpec((D, E), lambda i: (0, 0)),
            pl.BlockSpec((1, E), lambda i: (0, 0)),
        ],
        out_specs=[
            pl.BlockSpec((TB, K), lambda i: (i, 0)),
            pl.BlockSpec((TB, K), lambda i: (i, 0)),
            pl.BlockSpec((1, E), lambda i: (0, 0)),
            pl.BlockSpec((1, E), lambda i: (0, 0)),
            pl.BlockSpec((1, 1), lambda i: (0, 0)),
            pl.BlockSpec((1, 1), lambda i: (0, 0)),
        ],
        out_shape=[
            jax.ShapeDtypeStruct((T, K), jnp.int32),
            jax.ShapeDtypeStruct((T, K), jnp.float32),
            jax.ShapeDtypeStruct((1, E), jnp.float32),
            jax.ShapeDtypeStruct((1, E), jnp.float32),
            jax.ShapeDtypeStruct((1, 1), jnp.float32),
            jax.ShapeDtypeStruct((1, 1), jnp.float32),
        ],
    )(x, rW, rb.reshape(1, E))

    counts = cnt[0].astype(jnp.int32)
    EH = E // 2
    MAXB_H = N // B + EH
    NPAD_H = MAXB_H * B
    ZROW = NPAD_H
    NC, NS = 2, 16
    NW = NC * NS
    RPT = NPAD_H // NW
    NPT = N // NW
    JPT = NPT // 16
    CH = 32
    NCH = RPT // CH
    TPT = T // NW
    GCH = 128
    NG = NPT // GCH
    mesh = plsc.VectorSubcoreMesh(core_axis_name="c", subcore_axis_name="s",
                                  num_cores=NC, num_subcores=NS)

    @functools.partial(
        pl.kernel,
        out_type=[
            jax.ShapeDtypeStruct((NPAD_H, D), jnp.float32),
            jax.ShapeDtypeStruct((NPAD_H,), jnp.float32),
            jax.ShapeDtypeStruct((N,), jnp.int32),
        ],
        mesh=mesh,
        compiler_params=pltpu.CompilerParams(needs_layout_passes=False),
        scratch_types=[
            pltpu.VMEM((N // 2,), jnp.int32),
            pltpu.VMEM((N // 2,), jnp.float32),
            pltpu.VMEM((128,), jnp.int32),
            pltpu.VMEM((RPT,), jnp.int32),
            pltpu.VMEM((RPT,), jnp.float32),
            pltpu.VMEM((NPT,), jnp.int32),
            pltpu.VMEM((2, CH, D), jnp.float32),
            pltpu.SemaphoreType.DMA,
            pltpu.SemaphoreType.DMA,
            pltpu.SemaphoreType.DMA,
            pltpu.SemaphoreType.DMA,
        ],
    )
    def _dispatch(x_hbm, er_hbm, gate_hbm, offs_hbm,
                  xs_hbm, gs_hbm, p_hbm,
                  erv, gv, ov, stl, gsl, plv, buf, g0, g1, o0, o1):
        wid = lax.axis_index("s") * NC + lax.axis_index("c")
        base = wid * RPT
        with jax.named_scope("disp_meta"):
            pltpu.sync_copy(offs_hbm, ov)

            lane0 = lax.broadcasted_iota(jnp.int32, (16,), 0)
            zf = jnp.zeros((16,), jnp.float32)

            def zbody(i, _):
                stl[pl.ds(i * 16, 16)] = jnp.bitwise_and(
                    base + i * 16 + lane0, T - 1)
                gsl[pl.ds(i * 16, 16)] = zf
                return 0
            lax.fori_loop(0, RPT // 16, zbody, 0)

        lane = lax.broadcasted_iota(jnp.int32, (16,), 0)
        JH = N // 32

        def sbody(j, _):
            sl = pl.ds(jnp.bitwise_and(j, JH - 1) * 16, 16)
            er16 = erv[sl]
            e16 = lax.shift_right_logical(er16, 16)
            r16 = jnp.bitwise_and(er16, 65535)
            raw = plsc.load_gather(ov, [e16]) + r16
            p16 = jnp.where(raw < ZROW, raw,
                            ZROW + jnp.bitwise_and(r16, B - 1))
            rel = p16 - base
            own = (p16 >= base) & (p16 < base + RPT)
            tok16 = (j * 16 + lane) // K
            plsc.store_scatter(stl, [rel], tok16, mask=own)
            plsc.store_scatter(gsl, [rel], gv[sl], mask=own)

            @pl.when((j >= wid * JPT) & (j < (wid + 1) * JPT))
            def _own_p():
                plv[pl.ds((j - wid * JPT) * 16, 16)] = p16
            return 0
        with jax.named_scope("disp_scan"):
            for h in range(2):
                pltpu.sync_copy(er_hbm.at[pl.ds(h * N // 2, N // 2)], erv)
                pltpu.sync_copy(gate_hbm.at[pl.ds(h * N // 2, N // 2)], gv)
                lax.fori_loop(h * JH, (h + 1) * JH, sbody, 0)

        with jax.named_scope("disp_flush"):
            pltpu.sync_copy(plv, p_hbm.at[pl.ds(wid * NPT, NPT)])
            pltpu.sync_copy(gsl, gs_hbm.at[pl.ds(base, RPT)])

        gsem = [g0, g1]
        osem = [o0, o1]

        def gather(c, b):
            return pltpu.async_copy(
                x_hbm.at[stl.at[pl.ds(c * CH, CH)]], buf.at[b], gsem[b])

        with jax.named_scope("disp_gather"):
            gdesc = [gather(0, 0), None]
            odesc = [None, None]
            for c in range(NCH):
                b = c & 1
                if c + 1 < NCH:
                    if odesc[1 - b] is not None:
                        odesc[1 - b].wait()
                    gdesc[1 - b] = gather(c + 1, 1 - b)
                gdesc[b].wait()
                odesc[b] = pltpu.async_copy(
                    buf.at[b], xs_hbm.at[pl.ds(base + c * CH, CH)], osem[b])
            odesc[0].wait()
            odesc[1].wait()

    @functools.partial(
        pl.kernel,
        out_type=jax.ShapeDtypeStruct((T, C2), jnp.float32),
        mesh=mesh,
        compiler_params=pltpu.CompilerParams(needs_layout_passes=False),
        scratch_types=[
            pltpu.VMEM((NPT,), jnp.int32),
            pltpu.VMEM((GCH, C2), jnp.float32),
            pltpu.VMEM((TPT, C2), jnp.float32),
            pltpu.SemaphoreType.DMA,
        ],
    )
    def _combine(eo_hbm, p_hbm, out_hbm, pv, rows, outv, sem):
        wid = lax.axis_index("s") * NC + lax.axis_index("c")
        pltpu.sync_copy(p_hbm.at[pl.ds(wid * NPT, NPT)], pv)
        for c in range(NG):
            pltpu.async_copy(eo_hbm.at[pv.at[pl.ds(c * GCH, GCH)]], rows, sem).wait()

            def tbody(t, _):
                def cbody(q, _):
                    csl = pl.ds(q * 16, 16)
                    a = rows[t * K + 0, csl]
                    for k in range(1, K):
                        a = a + rows[t * K + k, csl]
                    outv[c * (GCH // K) + t, csl] = a
                    return 0
                lax.fori_loop(0, C2 // 16, cbody, 0)
                return 0
            lax.fori_loop(0, GCH // K, tbody, 0)
        pltpu.sync_copy(outv, out_hbm.at[pl.ds(wid * TPT, TPT)])

    def _half(e_lo):
        counts_h = counts[e_lo:e_lo + EH]
        pc = ((counts_h + B - 1) // B) * B
        nb = pc // B
        cum_nb = jnp.cumsum(nb)
        offs_h = (jnp.cumsum(pc) - pc).astype(jnp.int32)
        offs_full = jnp.full((128,), ZROW, jnp.int32)
        offs_full = lax.dynamic_update_slice(offs_full, offs_h, (e_lo,))
        total_nb = cum_nb[-1]
        bi = jnp.arange(MAXB_H + 1, dtype=jnp.int32)
        be = e_lo + jnp.searchsorted(cum_nb, bi, side="right").astype(jnp.int32)
        be = jnp.minimum(be, e_lo + EH - 1)
        valid = bi < total_nb
        last_b = jnp.maximum(total_nb - 1, 0)
        be_c = jnp.where(valid, be, be[last_b]).astype(jnp.int32)
        xb_i = jnp.where(valid, bi, last_b).astype(jnp.int32)
        valid_i = valid.astype(jnp.int32)

        xs, gs, ph = _dispatch(x, er.reshape(N), gate.reshape(N), offs_full)

        eo = pl.pallas_call(
            _expert_body,
            grid_spec=pltpu.PrefetchScalarGridSpec(
                num_scalar_prefetch=3,
                grid=(MAXB_H + 1,),
                in_specs=[
                    pl.BlockSpec((B, D), lambda i, bee, xbb, vll: (xbb[i], 0)),
                    pl.BlockSpec((B, 1), lambda i, bee, xbb, vll: (xbb[i], 0)),
                    pl.BlockSpec((1, D, H), lambda i, bee, xbb, vll: (bee[i], 0, 0)),
                    pl.BlockSpec((1, 1, H), lambda i, bee, xbb, vll: (bee[i], 0, 0)),
                    pl.BlockSpec((1, H, C), lambda i, bee, xbb, vll: (bee[i], 0, 0)),
                    pl.BlockSpec((1, 1, C), lambda i, bee, xbb, vll: (bee[i], 0, 0)),
                ],
                out_specs=pl.BlockSpec((B, C2), lambda i, bee, xbb, vll: (i, 0)),
                scratch_shapes=[pltpu.VMEM((D, H), jnp.bfloat16)],
            ),
            out_shape=jax.ShapeDtypeStruct((NPAD_H + B, C2), jnp.float32),
        )(be_c, xb_i, valid_i, xs,
          gs.reshape(NPAD_H, 1), W1, b1.reshape(E, 1, H), W2,
          b2.reshape(E, 1, C))

        return _combine(eo, ph)

    outp = _half(0) + _half(EH)
    return outp[:, :C], aux[0, 0]


# speedup vs baseline: 1.5458x; 1.5458x over previous
import functools

import jax
import jax.numpy as jnp
from jax import lax
from jax.experimental import pallas as pl
from jax.experimental.pallas import tpu as pltpu
from jax.experimental.pallas import tpu_sc as plsc

K = 8
B = 128


def _router_body(x_ref, rW_ref, rb_ref, er_ref, gate_ref, cnt_ref,
                 load_ref, z_ref, aux_ref, *, nblk, E, T, TB):
    i = pl.program_id(0)
    xb = x_ref[...]
    logits = jnp.dot(xb, rW_ref[...], preferred_element_type=jnp.float32) + rb_ref[...]
    mx = jnp.max(logits, axis=1, keepdims=True)
    ex = jnp.exp(logits - mx)
    se = jnp.sum(ex, axis=1, keepdims=True)
    probs = ex / se
    lse = mx + jnp.log(se)

    iota = lax.broadcasted_iota(jnp.int32, probs.shape, 1)
    work = probs
    sels, vals, ohs = [], [], []
    for _ in range(K):
        mj = jnp.max(work, axis=1, keepdims=True)
        ismax = work == mj
        sel = jnp.min(jnp.where(ismax, iota, E), axis=1, keepdims=True)
        onehot = iota == sel
        sels.append(sel)
        vals.append(mj)
        ohs.append(onehot)
        work = jnp.where(onehot, -jnp.inf, work)
    topv = jnp.concatenate(vals, axis=1)
    ssum = jnp.sum(topv, axis=1, keepdims=True)
    gates = topv / ssum
    gate_ref[...] = gates

    @pl.when(i == 0)
    def _init():
        cnt_ref[...] = jnp.zeros_like(cnt_ref)
        load_ref[...] = jnp.zeros_like(load_ref)
        z_ref[...] = jnp.zeros_like(z_ref)

    M = ohs[0].astype(jnp.float32)
    for oh in ohs[1:]:
        M = M + oh.astype(jnp.float32)
    ltri = (lax.broadcasted_iota(jnp.int32, (TB, TB), 0)
            > lax.broadcasted_iota(jnp.int32, (TB, TB), 1)).astype(jnp.float32)
    csum = jnp.dot(ltri, M, preferred_element_type=jnp.float32)
    tot = csum + cnt_ref[...]
    ranks = [jnp.sum(jnp.where(oh, tot, 0.0), axis=1, keepdims=True) for oh in ohs]
    rank_i = jnp.concatenate(ranks, axis=1).astype(jnp.int32)
    er_ref[...] = jnp.concatenate(sels, axis=1) * 65536 + rank_i

    cnt_ref[...] += jnp.sum(M, axis=0)[None, :]

    maskb = jnp.zeros_like(probs)
    for kk, oh in enumerate(ohs):
        maskb = maskb + jnp.where(oh, gates[:, kk:kk + 1], 0.0)
    load_ref[...] += jnp.sum(maskb, axis=0)[None, :]
    z_ref[...] += jnp.reshape(jnp.sum(lse * lse), (1, 1))

    @pl.when(i == nblk - 1)
    def _fin():
        load = load_ref[...] / T
        lb = 0.1 * jnp.sum((load - 1.0 / E) ** 2)
        aux_ref[...] = lb + 0.1 * z_ref[...] / T


def _expert_body(be_ref, xb_ref, vl_ref, xs_ref, gs_ref, W1_ref, b1_ref,
                 W2_ref, b2_ref, eo_ref, w1c_ref):
    i = pl.program_id(0)

    @pl.when((vl_ref[i] == 1)
             & ((i == 0) | (be_ref[i] != be_ref[jnp.maximum(i - 1, 0)])))
    def _cache():
        w1c_ref[...] = W1_ref[0].astype(jnp.bfloat16)

    @pl.when(vl_ref[i] == 1)
    def _run():
        xb = xs_ref[...].astype(jnp.bfloat16)
        h = jnp.maximum(
            jnp.dot(xb, w1c_ref[...], preferred_element_type=jnp.float32)
            + b1_ref[0], 0.0)
        w2 = W2_ref[0].astype(jnp.bfloat16)
        eo = jnp.dot(h.astype(jnp.bfloat16), w2,
                     preferred_element_type=jnp.float32) + b2_ref[0]
        eo = eo * gs_ref[...]
        pad = jnp.zeros((eo.shape[0], eo_ref.shape[1] - eo.shape[1]), jnp.float32)
        eo_ref[...] = jnp.concatenate([eo, pad], axis=1)


def kernel(x, rW, rb, W1, b1, W2, b2):
    T, D = x.shape
    E = rW.shape[1]
    H = W1.shape[2]
    C = W2.shape[2]
    C2 = 256
    TB = 256
    nblk = T // TB
    N = T * K
    MAXB = N // B + E
    NPAD = MAXB * B

    er, gate, cnt, _load, _z, aux = pl.pallas_call(
        functools.partial(_router_body, nblk=nblk, E=E, T=T, TB=TB),
        grid=(nblk,),
        in_specs=[
            pl.BlockSpec((TB, D), lambda i: (i, 0)),
            pl.BlockSpec((D, E), lambda i: (0, 0)),
            pl.BlockSpec((1, E), lambda i: (0, 0)),
        ],
        out_specs=[
            pl.BlockSpec((TB, K), lambda i: (i, 0)),
            pl.BlockSpec((TB, K), lambda i: (i, 0)),
            pl.BlockSpec((1, E), lambda i: (0, 0)),
            pl.BlockSpec((1, E), lambda i: (0, 0)),
            pl.BlockSpec((1, 1), lambda i: (0, 0)),
            pl.BlockSpec((1, 1), lambda i: (0, 0)),
        ],
        out_shape=[
            jax.ShapeDtypeStruct((T, K), jnp.int32),
            jax.ShapeDtypeStruct((T, K), jnp.float32),
            jax.ShapeDtypeStruct((1, E), jnp.float32),
            jax.ShapeDtypeStruct((1, E), jnp.float32),
            jax.ShapeDtypeStruct((1, 1), jnp.float32),
            jax.ShapeDtypeStruct((1, 1), jnp.float32),
        ],
    )(x, rW, rb.reshape(1, E))

    counts = cnt[0].astype(jnp.int32)
    pc = ((counts + B - 1) // B) * B
    nb = pc // B
    cum_nb = jnp.cumsum(nb)
    offs = jnp.pad((jnp.cumsum(pc) - pc).astype(jnp.int32), (0, 128 - E))
    total_nb = cum_nb[-1]
    bi = jnp.arange(MAXB, dtype=jnp.int32)
    be = jnp.searchsorted(cum_nb, bi, side="right").astype(jnp.int32)
    valid = bi < total_nb
    last_b = jnp.maximum(total_nb - 1, 0)
    be_c = jnp.where(valid, be, be[last_b]).astype(jnp.int32)
    xb_i = jnp.where(valid, bi, last_b).astype(jnp.int32)
    valid_i = valid.astype(jnp.int32)

    NC, NS = 2, 16
    NW = NC * NS
    RPT = NPAD // NW
    NPT = N // NW
    JPT = NPT // 16
    CH = 32
    NCH = RPT // CH
    TPT = T // NW
    mesh = plsc.VectorSubcoreMesh(core_axis_name="c", subcore_axis_name="s",
                                  num_cores=NC, num_subcores=NS)

    @functools.partial(
        pl.kernel,
        out_type=[
            jax.ShapeDtypeStruct((NPAD, D), jnp.float32),
            jax.ShapeDtypeStruct((NPAD,), jnp.float32),
            jax.ShapeDtypeStruct((N,), jnp.int32),
        ],
        mesh=mesh,
        compiler_params=pltpu.CompilerParams(needs_layout_passes=False),
        scratch_types=[
            pltpu.VMEM((N // 2,), jnp.int32),
            pltpu.VMEM((N // 2,), jnp.float32),
            pltpu.VMEM((128,), jnp.int32),
            pltpu.VMEM((RPT,), jnp.int32),
            pltpu.VMEM((RPT,), jnp.float32),
            pltpu.VMEM((NPT,), jnp.int32),
            pltpu.VMEM((2, CH, D), jnp.float32),
            pltpu.SemaphoreType.DMA,
            pltpu.SemaphoreType.DMA,
            pltpu.SemaphoreType.DMA,
            pltpu.SemaphoreType.DMA,
        ],
    )
    def _dispatch(x_hbm, er_hbm, gate_hbm, offs_hbm,
                  xs_hbm, gs_hbm, p_hbm,
                  erv, gv, ov, stl, gsl, plv, buf, g0, g1, o0, o1):
        wid = lax.axis_index("s") * NC + lax.axis_index("c")
        base = wid * RPT
        with jax.named_scope("disp_meta"):
            pltpu.sync_copy(offs_hbm, ov)

            lane0 = lax.broadcasted_iota(jnp.int32, (16,), 0)
            zf = jnp.zeros((16,), jnp.float32)

            def zbody(i, _):
                stl[pl.ds(i * 16, 16)] = jnp.bitwise_and(
                    base + i * 16 + lane0, T - 1)
                gsl[pl.ds(i * 16, 16)] = zf
                return 0
            lax.fori_loop(0, RPT // 16, zbody, 0)

        lane = lax.broadcasted_iota(jnp.int32, (16,), 0)
        JH = N // 32

        def sbody(j, _):
            sl = pl.ds(jnp.bitwise_and(j, JH - 1) * 16, 16)
            er16 = erv[sl]
            e16 = lax.shift_right_logical(er16, 16)
            r16 = jnp.bitwise_and(er16, 65535)
            p16 = plsc.load_gather(ov, [e16]) + r16
            rel = p16 - base
            own = (p16 >= base) & (p16 < base + RPT)
            tok16 = (j * 16 + lane) // K
            plsc.store_scatter(stl, [rel], tok16, mask=own)
            plsc.store_scatter(gsl, [rel], gv[sl], mask=own)

            @pl.when((j >= wid * JPT) & (j < (wid + 1) * JPT))
            def _own_p():
                plv[pl.ds((j - wid * JPT) * 16, 16)] = p16
            return 0
        with jax.named_scope("disp_scan"):
            for h in range(2):
                pltpu.sync_copy(er_hbm.at[pl.ds(h * N // 2, N // 2)], erv)
                pltpu.sync_copy(gate_hbm.at[pl.ds(h * N // 2, N // 2)], gv)
                lax.fori_loop(h * JH, (h + 1) * JH, sbody, 0)

        with jax.named_scope("disp_flush"):
            pltpu.sync_copy(plv, p_hbm.at[pl.ds(wid * NPT, NPT)])
            pltpu.sync_copy(gsl, gs_hbm.at[pl.ds(base, RPT)])

        gsem = [g0, g1]
        osem = [o0, o1]

        def gather(c, b):
            return pltpu.async_copy(
                x_hbm.at[stl.at[pl.ds(c * CH, CH)]], buf.at[b], gsem[b])

        with jax.named_scope("disp_gather"):
            gdesc = [gather(0, 0), None]
            odesc = [None, None]
            for c in range(NCH):
                b = c & 1
                if c + 1 < NCH:
                    if odesc[1 - b] is not None:
                        odesc[1 - b].wait()
                    gdesc[1 - b] = gather(c + 1, 1 - b)
                gdesc[b].wait()
                odesc[b] = pltpu.async_copy(
                    buf.at[b], xs_hbm.at[pl.ds(base + c * CH, CH)], osem[b])
            odesc[0].wait()
            odesc[1].wait()

    xs, gs, p = _dispatch(x, er.reshape(N), gate.reshape(N), offs)

    eo = pl.pallas_call(
        _expert_body,
        grid_spec=pltpu.PrefetchScalarGridSpec(
            num_scalar_prefetch=3,
            grid=(MAXB,),
            in_specs=[
                pl.BlockSpec((B, D), lambda i, bee, xbb, vll: (xbb[i], 0)),
                pl.BlockSpec((B, 1), lambda i, bee, xbb, vll: (xbb[i], 0)),
                pl.BlockSpec((1, D, H), lambda i, bee, xbb, vll: (bee[i], 0, 0)),
                pl.BlockSpec((1, 1, H), lambda i, bee, xbb, vll: (bee[i], 0, 0)),
                pl.BlockSpec((1, H, C), lambda i, bee, xbb, vll: (bee[i], 0, 0)),
                pl.BlockSpec((1, 1, C), lambda i, bee, xbb, vll: (bee[i], 0, 0)),
            ],
            out_specs=pl.BlockSpec((B, C2), lambda i, bee, xbb, vll: (xbb[i], 0)),
            scratch_shapes=[pltpu.VMEM((D, H), jnp.bfloat16)],
        ),
        out_shape=jax.ShapeDtypeStruct((NPAD, C2), jnp.float32),
    )(be_c, xb_i, valid_i, xs,
      gs.reshape(NPAD, 1), W1, b1.reshape(E, 1, H), W2, b2.reshape(E, 1, C))

    GCH = 128
    NG = NPT // GCH

    @functools.partial(
        pl.kernel,
        out_type=jax.ShapeDtypeStruct((T, C2), jnp.float32),
        mesh=mesh,
        compiler_params=pltpu.CompilerParams(needs_layout_passes=False),
        scratch_types=[
            pltpu.VMEM((NPT,), jnp.int32),
            pltpu.VMEM((GCH, C2), jnp.float32),
            pltpu.VMEM((TPT, C2), jnp.float32),
            pltpu.SemaphoreType.DMA,
        ],
    )
    def _combine(eo_hbm, p_hbm, out_hbm, pv, rows, outv, sem):
        wid = lax.axis_index("s") * NC + lax.axis_index("c")
        pltpu.sync_copy(p_hbm.at[pl.ds(wid * NPT, NPT)], pv)
        for c in range(NG):
            pltpu.async_copy(eo_hbm.at[pv.at[pl.ds(c * GCH, GCH)]], rows, sem).wait()

            def tbody(t, _):
                def cbody(q, _):
                    csl = pl.ds(q * 16, 16)
                    a = rows[t * K + 0, csl]
                    for k in range(1, K):
                        a = a + rows[t * K + k, csl]
                    outv[c * (GCH // K) + t, csl] = a
                    return 0
                lax.fori_loop(0, C2 // 16, cbody, 0)
                return 0
            lax.fori_loop(0, GCH // K, tbody, 0)
        pltpu.sync_copy(outv, out_hbm.at[pl.ds(wid * TPT, TPT)])

    outp = _combine(eo, p)
    return outp[:, :C], aux[0, 0]


# speedup vs baseline: 1.5539x; 1.0052x over previous
import functools

import jax
import jax.numpy as jnp
from jax import lax
from jax.experimental import pallas as pl
from jax.experimental.pallas import tpu as pltpu
from jax.experimental.pallas import tpu_sc as plsc

K = 8
B = 128


def _router_body(x_ref, rW_ref, rb_ref, er_ref, gate_ref, cnt_ref,
                 load_ref, z_ref, aux_ref, *, nblk, E, T, TB):
    i = pl.program_id(0)
    xb = x_ref[...]
    logits = jnp.dot(xb, rW_ref[...], preferred_element_type=jnp.float32) + rb_ref[...]
    mx = jnp.max(logits, axis=1, keepdims=True)
    ex = jnp.exp(logits - mx)
    se = jnp.sum(ex, axis=1, keepdims=True)
    probs = ex / se
    lse = mx + jnp.log(se)

    iota = lax.broadcasted_iota(jnp.int32, probs.shape, 1)
    work = probs
    sels, vals, ohs = [], [], []
    for _ in range(K):
        mj = jnp.max(work, axis=1, keepdims=True)
        ismax = work == mj
        sel = jnp.min(jnp.where(ismax, iota, E), axis=1, keepdims=True)
        onehot = iota == sel
        sels.append(sel)
        vals.append(mj)
        ohs.append(onehot)
        work = jnp.where(onehot, -jnp.inf, work)
    topv = jnp.concatenate(vals, axis=1)
    ssum = jnp.sum(topv, axis=1, keepdims=True)
    gates = topv / ssum
    gate_ref[...] = gates

    @pl.when(i == 0)
    def _init():
        cnt_ref[...] = jnp.zeros_like(cnt_ref)
        load_ref[...] = jnp.zeros_like(load_ref)
        z_ref[...] = jnp.zeros_like(z_ref)

    M = ohs[0].astype(jnp.float32)
    for oh in ohs[1:]:
        M = M + oh.astype(jnp.float32)
    ltri = (lax.broadcasted_iota(jnp.int32, (TB, TB), 0)
            > lax.broadcasted_iota(jnp.int32, (TB, TB), 1)).astype(jnp.float32)
    csum = jnp.dot(ltri, M, preferred_element_type=jnp.float32)
    tot = csum + cnt_ref[...]
    ranks = [jnp.sum(jnp.where(oh, tot, 0.0), axis=1, keepdims=True) for oh in ohs]
    rank_i = jnp.concatenate(ranks, axis=1).astype(jnp.int32)
    er_ref[...] = jnp.concatenate(sels, axis=1) * 65536 + rank_i

    cnt_ref[...] += jnp.sum(M, axis=0)[None, :]

    maskb = jnp.zeros_like(probs)
    for kk, oh in enumerate(ohs):
        maskb = maskb + jnp.where(oh, gates[:, kk:kk + 1], 0.0)
    load_ref[...] += jnp.sum(maskb, axis=0)[None, :]
    z_ref[...] += jnp.reshape(jnp.sum(lse * lse), (1, 1))

    @pl.when(i == nblk - 1)
    def _fin():
        load = load_ref[...] / T
        lb = 0.1 * jnp.sum((load - 1.0 / E) ** 2)
        aux_ref[...] = lb + 0.1 * z_ref[...] / T


def _expert_body(be_ref, xb_ref, vl_ref, xs_ref, gs_ref, W1_ref, b1_ref,
                 W2_ref, b2_ref, eo_ref, w1c_ref):
    i = pl.program_id(0)

    @pl.when((vl_ref[i] == 1)
             & ((i == 0) | (be_ref[i] != be_ref[jnp.maximum(i - 1, 0)])))
    def _cache():
        w1c_ref[...] = W1_ref[0].astype(jnp.bfloat16)

    @pl.when(vl_ref[i] == 1)
    def _run():
        xb = xs_ref[...].astype(jnp.bfloat16)
        h = jnp.maximum(
            jnp.dot(xb, w1c_ref[...], preferred_element_type=jnp.float32)
            + b1_ref[0], 0.0)
        w2 = W2_ref[0].astype(jnp.bfloat16)
        eo = jnp.dot(h.astype(jnp.bfloat16), w2,
                     preferred_element_type=jnp.float32) + b2_ref[0]
        eo = eo * gs_ref[...]
        pad = jnp.zeros((eo.shape[0], eo_ref.shape[1] - eo.shape[1]), jnp.float32)
        eo_ref[...] = jnp.concatenate([eo, pad], axis=1)


def kernel(x, rW, rb, W1, b1, W2, b2):
    T, D = x.shape
    E = rW.shape[1]
    H = W1.shape[2]
    C = W2.shape[2]
    C2 = 256
    TB = 256
    nblk = T // TB
    N = T * K
    MAXB = N // B + E
    NPAD = MAXB * B

    er, gate, cnt, _load, _z, aux = pl.pallas_call(
        functools.partial(_router_body, nblk=nblk, E=E, T=T, TB=TB),
        grid=(nblk,),
        in_specs=[
            pl.BlockSpec((TB, D), lambda i: (i, 0)),
            pl.BlockSpec((D, E), lambda i: (0, 0)),
            pl.BlockSpec((1, E), lambda i: (0, 0)),
        ],
        out_specs=[
            pl.BlockSpec((TB, K), lambda i: (i, 0)),
            pl.BlockSpec((TB, K), lambda i: (i, 0)),
            pl.BlockSpec((1, E), lambda i: (0, 0)),
            pl.BlockSpec((1, E), lambda i: (0, 0)),
            pl.BlockSpec((1, 1), lambda i: (0, 0)),
            pl.BlockSpec((1, 1), lambda i: (0, 0)),
        ],
        out_shape=[
            jax.ShapeDtypeStruct((T, K), jnp.int32),
            jax.ShapeDtypeStruct((T, K), jnp.float32),
            jax.ShapeDtypeStruct((1, E), jnp.float32),
            jax.ShapeDtypeStruct((1, E), jnp.float32),
            jax.ShapeDtypeStruct((1, 1), jnp.float32),
            jax.ShapeDtypeStruct((1, 1), jnp.float32),
        ],
    )(x, rW, rb.reshape(1, E))

    counts = cnt[0].astype(jnp.int32)
    pc = ((counts + B - 1) // B) * B
    nb = pc // B
    cum_nb = jnp.cumsum(nb)
    offs = jnp.pad((jnp.cumsum(pc) - pc).astype(jnp.int32), (0, 128 - E))
    total_nb = cum_nb[-1]
    bi = jnp.arange(MAXB, dtype=jnp.int32)
    be = jnp.searchsorted(cum_nb, bi, side="right").astype(jnp.int32)
    valid = bi < total_nb
    last_b = jnp.maximum(total_nb - 1, 0)
    be_c = jnp.where(valid, be, be[last_b]).astype(jnp.int32)
    xb_i = jnp.where(valid, bi, last_b).astype(jnp.int32)
    valid_i = valid.astype(jnp.int32)

    NC, NS = 2, 16
    NW = NC * NS
    RPT = NPAD // NW
    NPT = N // NW
    JPT = NPT // 16
    CH = 32
    NCH = RPT // CH
    TPT = T // NW
    mesh = plsc.VectorSubcoreMesh(core_axis_name="c", subcore_axis_name="s",
                                  num_cores=NC, num_subcores=NS)

    @functools.partial(
        pl.kernel,
        out_type=[
            jax.ShapeDtypeStruct((NPAD, D), jnp.float32),
            jax.ShapeDtypeStruct((NPAD,), jnp.float32),
            jax.ShapeDtypeStruct((N,), jnp.int32),
        ],
        mesh=mesh,
        compiler_params=pltpu.CompilerParams(needs_layout_passes=False),
        scratch_types=[
            pltpu.VMEM((N // 2,), jnp.int32),
            pltpu.VMEM((N // 2,), jnp.float32),
            pltpu.VMEM((128,), jnp.int32),
            pltpu.VMEM((RPT,), jnp.int32),
            pltpu.VMEM((RPT,), jnp.float32),
            pltpu.VMEM((NPT,), jnp.int32),
            pltpu.VMEM((2, CH, D), jnp.float32),
            pltpu.SemaphoreType.DMA,
            pltpu.SemaphoreType.DMA,
            pltpu.SemaphoreType.DMA,
            pltpu.SemaphoreType.DMA,
        ],
    )
    def _dispatch(x_hbm, er_hbm, gate_hbm, offs_hbm,
                  xs_hbm, gs_hbm, p_hbm,
                  erv, gv, ov, stl, gsl, plv, buf, g0, g1, o0, o1):
        wid = lax.axis_index("s") * NC + lax.axis_index("c")
        base = wid * RPT
        with jax.named_scope("disp_meta"):
            pltpu.sync_copy(offs_hbm, ov)

            lane0 = lax.broadcasted_iota(jnp.int32, (16,), 0)
            zf = jnp.zeros((16,), jnp.float32)

            def zbody(i, _):
                stl[pl.ds(i * 16, 16)] = jnp.bitwise_and(
                    base + i * 16 + lane0, T - 1)
                gsl[pl.ds(i * 16, 16)] = zf
                return 0
            lax.fori_loop(0, RPT // 16, zbody, 0)

        lane = lax.broadcasted_iota(jnp.int32, (16,), 0)
        JH = N // 32

        def sbody(j, _):
            sl = pl.ds(jnp.bitwise_and(j, JH - 1) * 16, 16)
            er16 = erv[sl]
            e16 = lax.shift_right_logical(er16, 16)
            r16 = jnp.bitwise_and(er16, 65535)
            p16 = plsc.load_gather(ov, [e16]) + r16
            rel = p16 - base
            own = (p16 >= base) & (p16 < base + RPT)
            tok16 = (j * 16 + lane) // K
            plsc.store_scatter(stl, [rel], tok16, mask=own)
            plsc.store_scatter(gsl, [rel], gv[sl], mask=own)

            @pl.when((j >= wid * JPT) & (j < (wid + 1) * JPT))
            def _own_p():
                plv[pl.ds((j - wid * JPT) * 16, 16)] = p16
            return 0
        with jax.named_scope("disp_scan"):
            for h in range(2):
                pltpu.sync_copy(er_hbm.at[pl.ds(h * N // 2, N // 2)], erv)
                pltpu.sync_copy(gate_hbm.at[pl.ds(h * N // 2, N // 2)], gv)
                lax.fori_loop(h * JH, (h + 1) * JH, sbody, 0)

        with jax.named_scope("disp_flush"):
            pltpu.sync_copy(plv, p_hbm.at[pl.ds(wid * NPT, NPT)])
            pltpu.sync_copy(gsl, gs_hbm.at[pl.ds(base, RPT)])

        gsem = [g0, g1]
        osem = [o0, o1]

        def gather(c, b):
            return pltpu.async_copy(
                x_hbm.at[stl.at[pl.ds(c * CH, CH)]], buf.at[b], gsem[b])

        with jax.named_scope("disp_gather"):
            gdesc = [gather(0, 0), None]
            odesc = [None, None]
            for c in range(NCH):
                b = c & 1
                if c + 1 < NCH:
                    if odesc[1 - b] is not None:
                        odesc[1 - b].wait()
                    gdesc[1 - b] = gather(c + 1, 1 - b)
                gdesc[b].wait()
                odesc[b] = pltpu.async_copy(
                    buf.at[b], xs_hbm.at[pl.ds(base + c * CH, CH)], osem[b])
            odesc[0].wait()
            odesc[1].wait()

    xs, gs, p = _dispatch(x, er.reshape(N), gate.reshape(N), offs)

    eo = pl.pallas_call(
        _expert_body,
        grid_spec=pltpu.PrefetchScalarGridSpec(
            num_scalar_prefetch=3,
            grid=(MAXB,),
            in_specs=[
                pl.BlockSpec((B, D), lambda i, bee, xbb, vll: (xbb[i], 0)),
                pl.BlockSpec((B, 1), lambda i, bee, xbb, vll: (xbb[i], 0)),
                pl.BlockSpec((1, D, H), lambda i, bee, xbb, vll: (bee[i], 0, 0)),
                pl.BlockSpec((1, 1, H), lambda i, bee, xbb, vll: (bee[i], 0, 0)),
                pl.BlockSpec((1, H, C), lambda i, bee, xbb, vll: (bee[i], 0, 0)),
                pl.BlockSpec((1, 1, C), lambda i, bee, xbb, vll: (bee[i], 0, 0)),
            ],
            out_specs=pl.BlockSpec((B, C2), lambda i, bee, xbb, vll: (xbb[i], 0)),
            scratch_shapes=[pltpu.VMEM((D, H), jnp.bfloat16)],
        ),
        out_shape=jax.ShapeDtypeStruct((NPAD, C2), jnp.float32),
    )(be_c, xb_i, valid_i, xs,
      gs.reshape(NPAD, 1), W1, b1.reshape(E, 1, H), W2, b2.reshape(E, 1, C))

    GCH = 128
    NG = NPT // GCH

    @functools.partial(
        pl.kernel,
        out_type=jax.ShapeDtypeStruct((T, C2), jnp.float32),
        mesh=mesh,
        compiler_params=pltpu.CompilerParams(needs_layout_passes=False),
        scratch_types=[
            pltpu.VMEM((NPT,), jnp.int32),
            pltpu.VMEM((GCH, C2), jnp.float32),
            pltpu.VMEM((TPT, C2), jnp.float32),
            pltpu.SemaphoreType.DMA,
        ],
    )
    def _combine(eo_hbm, p_hbm, out_hbm, pv, rows, outv, sem):
        wid = lax.axis_index("s") * NC + lax.axis_index("c")
        pltpu.sync_copy(p_hbm.at[pl.ds(wid * NPT, NPT)], pv)
        for c in range(NG):
            pltpu.async_copy(eo_hbm.at[pv.at[pl.ds(c * GCH, GCH)]], rows, sem).wait()

            def tbody(t, _):
                def cbody(q, _):
                    csl = pl.ds(q * 16, 16)
                    a = rows[t * K + 0, csl]
                    for k in range(1, K):
                        a = a + rows[t * K + k, csl]
                    outv[c * (GCH // K) + t, csl] = a
                    return 0
                lax.fori_loop(0, 192 // 16, cbody, 0)
                return 0
            lax.fori_loop(0, GCH // K, tbody, 0)
        pltpu.sync_copy(outv, out_hbm.at[pl.ds(wid * TPT, TPT)])

    outp = _combine(eo, p)
    return outp[:, :C], aux[0, 0]
